# baseline (speedup 1.0000x reference)
import functools
import math

import jax
import jax.numpy as jnp
from jax import lax
from jax.experimental import pallas as pl
from jax.experimental.pallas import tpu as pltpu
from jax.experimental.pallas import tpu_sc as plsc

T, D, E, K, FF = 2048, 768, 64, 2, 1536
RH = 2 * D
C = int(math.ceil(T * K / E * 1.25))
EC = E * C
TB = 256
SB = 512


def _gelu(v):
    return 0.5 * v * (1.0 + lax.erf(v * (1.0 / math.sqrt(2.0))))


def _router_body(x_ref, w1_ref, b1_ref, w2_ref, b2_ref,
                 i1_ref, i2_ref, g1_ref, g2_ref):
    xb = x_ref[...]
    rh = _gelu(jnp.dot(xb, w1_ref[...], preferred_element_type=jnp.float32)
               + b1_ref[...])
    logits = (jnp.dot(rh, w2_ref[...], preferred_element_type=jnp.float32)
              + b2_ref[...])
    iota = lax.broadcasted_iota(jnp.int32, (TB, E), 1)
    l1 = jnp.max(logits, axis=-1, keepdims=True)
    a1 = jnp.min(jnp.where(logits == l1, iota, E), axis=-1, keepdims=True)
    masked = jnp.where(iota == a1, -jnp.inf, logits)
    l2 = jnp.max(masked, axis=-1, keepdims=True)
    a2 = jnp.min(jnp.where(masked == l2, iota, E), axis=-1, keepdims=True)
    g1 = 1.0 / (1.0 + jnp.exp(l2 - l1))
    i1_ref[...] = a1
    i2_ref[...] = a2
    g1_ref[...] = g1
    g2_ref[...] = 1.0 - g1


def _router(x, rW1, rb1, rW2, rb2):
    n = T // TB
    out_shapes = (
        jax.ShapeDtypeStruct((T, 1), jnp.int32),
        jax.ShapeDtypeStruct((T, 1), jnp.int32),
        jax.ShapeDtypeStruct((T, 1), jnp.float32),
        jax.ShapeDtypeStruct((T, 1), jnp.float32),
    )
    return pl.pallas_call(
        _router_body,
        grid=(n,),
        in_specs=[
            pl.BlockSpec((TB, D), lambda i: (i, 0)),
            pl.BlockSpec((D, RH), lambda i: (0, 0)),
            pl.BlockSpec((RH,), lambda i: (0,)),
            pl.BlockSpec((RH, E), lambda i: (0, 0)),
            pl.BlockSpec((E,), lambda i: (0,)),
        ],
        out_specs=tuple(pl.BlockSpec((TB, 1), lambda i: (i, 0))
                        for _ in range(4)),
        out_shape=out_shapes,
    )(x, rW1, rb1, rW2, rb2)


def _pos_body(i1_ref, i2_ref, s1_ref, s2_ref):
    i1 = i1_ref[...]
    i2 = i2_ref[...]
    iota = lax.broadcasted_iota(jnp.int32, (T, E), 1)
    oh1 = (i1 == iota).astype(jnp.int32)
    oh2 = (i2 == iota).astype(jnp.int32)
    s = oh1 + oh2
    k = 1
    while k < T:
        s = s + jnp.concatenate(
            [jnp.zeros((k, E), jnp.int32), s[:T - k]], axis=0)
        k *= 2
    pos1 = jnp.sum(oh1 * (s - oh1 - oh2), axis=1, keepdims=True)
    pos2 = jnp.sum(oh2 * (s - oh2), axis=1, keepdims=True)
    s1 = jnp.where(pos1 < C, i1 * C + pos1, EC)
    s2 = jnp.where(pos2 < C, i2 * C + pos2, EC)
    s1_ref[...] = s1
    s2_ref[...] = s2


def _positions(i1, i2):
    return pl.pallas_call(
        _pos_body,
        out_shape=(jax.ShapeDtypeStruct((T, 1), jnp.int32),
                   jax.ShapeDtypeStruct((T, 1), jnp.int32)),
    )(i1, i2)


def _disp_body(s1_ref, s2_ref, g1_ref, g2_ref, tok_ref, gate_ref):
    base = pl.program_id(0) * SB
    lane = base + lax.broadcasted_iota(jnp.int32, (1, SB), 1)
    tokf = lax.broadcasted_iota(jnp.int32, (T, 1), 0).astype(jnp.float32)
    m1 = (s1_ref[...] == lane).astype(jnp.float32)
    m2 = (s2_ref[...] == lane).astype(jnp.float32)
    dn = (((0,), (0,)), ((), ()))
    toki = lax.broadcasted_iota(jnp.int32, (T, 1), 0)
    tok_hi = (toki >> 7).astype(jnp.float32)
    tok_lo = (toki & 127).astype(jnp.float32)
    m12 = m1 + m2
    hi = lax.dot_general(m12, tok_hi, dn, preferred_element_type=jnp.float32)
    lo = lax.dot_general(m12, tok_lo, dn, preferred_element_type=jnp.float32)
    gate = (lax.dot_general(m1, g1_ref[...], dn,
                            preferred_element_type=jnp.float32)
            + lax.dot_general(m2, g2_ref[...], dn,
                              preferred_element_type=jnp.float32))
    tok_ref[...] = (hi.astype(jnp.int32) << 7) | lo.astype(jnp.int32)
    gate_ref[...] = gate


def _disp_table(s1, s2, g1, g2):
    full = pl.BlockSpec((T, 1), lambda i: (0, 0))
    return pl.pallas_call(
        _disp_body,
        grid=(EC // SB,),
        in_specs=[full, full, full, full],
        out_specs=(pl.BlockSpec((SB, 1), lambda i: (i, 0)),
                   pl.BlockSpec((SB, 1), lambda i: (i, 0))),
        out_shape=(jax.ShapeDtypeStruct((EC, 1), jnp.int32),
                   jax.ShapeDtypeStruct((EC, 1), jnp.float32)),
    )(s1, s2, g1, g2)


_NW = 32


def _sc_gather(x, disp_tok_flat):
    mesh = plsc.VectorSubcoreMesh(core_axis_name="c", subcore_axis_name="s")

    @functools.partial(
        pl.kernel, mesh=mesh,
        out_type=jax.ShapeDtypeStruct((EC, D), jnp.float32),
        scratch_types=[
            pltpu.VMEM((80,), jnp.int32),
            pltpu.VMEM((80,), jnp.int32),
            pltpu.VMEM((80, D), jnp.float32),
            pltpu.VMEM((80, D), jnp.float32),
            pltpu.SemaphoreType.DMA,
            pltpu.SemaphoreType.DMA,
        ],
    )
    def k(x_hbm, idx_hbm, xe_hbm, idx0_v, idx1_v, rows0, rows1, sem0, sem1):
        wid = lax.axis_index("s") * 2 + lax.axis_index("c")
        pltpu.sync_copy(idx_hbm.at[pl.ds(wid * 160, 80)], idx0_v)
        pltpu.sync_copy(idx_hbm.at[pl.ds(wid * 160 + 80, 80)], idx1_v)
        c0 = pltpu.async_copy(x_hbm.at[idx0_v], rows0, sem0)
        c1 = pltpu.async_copy(x_hbm.at[idx1_v], rows1, sem1)
        c0.wait()
        pltpu.sync_copy(rows0, xe_hbm.at[pl.ds(wid * 160, 80)])
        c1.wait()
        pltpu.sync_copy(rows1, xe_hbm.at[pl.ds(wid * 160 + 80, 80)])

    return k(x, disp_tok_flat)


def _sc_combine(ye, s1, s2):
    tpw = T // _NW
    mesh = plsc.VectorSubcoreMesh(core_axis_name="c", subcore_axis_name="s")

    @functools.partial(
        pl.kernel, mesh=mesh,
        out_type=jax.ShapeDtypeStruct((T, D), jnp.float32),
        scratch_types=[
            pltpu.VMEM((tpw,), jnp.int32),
            pltpu.VMEM((tpw,), jnp.int32),
            pltpu.VMEM((tpw, D), jnp.float32),
            pltpu.VMEM((tpw, D), jnp.float32),
            pltpu.SemaphoreType.DMA,
            pltpu.SemaphoreType.DMA,
        ],
    )
    def k(ye_hbm, s1_hbm, s2_hbm, out_hbm, i1_v, i2_v, r1, r2, sem0, sem1):
        wid = lax.axis_index("s") * 2 + lax.axis_index("c")
        base = wid * tpw
        pltpu.sync_copy(s1_hbm.at[pl.ds(base, tpw)], i1_v)
        pltpu.sync_copy(s2_hbm.at[pl.ds(base, tpw)], i2_v)
        c0 = pltpu.async_copy(ye_hbm.at[i1_v], r1, sem0)
        c1 = pltpu.async_copy(ye_hbm.at[i2_v], r2, sem1)
        c0.wait()
        c1.wait()

        def body(i, carry):
            for j in range(D // 16):
                sl = pl.ds(j * 16, 16)
                r1[i, sl] = r1[i, sl] + r2[i, sl]
            return carry

        lax.fori_loop(0, tpw, body, 0)
        pltpu.sync_copy(r1, out_hbm.at[pl.ds(base, tpw)])

    return k(ye, s1, s2)


def _ffn_body(xe_ref, w1_ref, b1_ref, w2_ref, b2_ref, gate_ref, ye_ref):
    e = pl.program_id(0)

    @pl.when(e < E)
    def _():
        xb = xe_ref[...]
        h = _gelu(jnp.dot(xb, w1_ref[0], preferred_element_type=jnp.float32)
                  + b1_ref[0])
        y = (jnp.dot(h, w2_ref[0], preferred_element_type=jnp.float32)
             + b2_ref[0])
        ye_ref[...] = y * gate_ref[...]

    @pl.when(e >= E)
    def _():
        ye_ref[...] = jnp.zeros((C, D), jnp.float32)


def _ffn(xe, eW1, eb1, eW2, eb2, disp_gate):
    wi = lambda e: (jnp.minimum(e, E - 1), 0, 0)
    return pl.pallas_call(
        _ffn_body,
        grid=(E + 1,),
        in_specs=[
            pl.BlockSpec((C, D), lambda e: (jnp.minimum(e, E - 1), 0)),
            pl.BlockSpec((1, D, FF), wi),
            pl.BlockSpec((1, 1, FF), wi),
            pl.BlockSpec((1, FF, D), wi),
            pl.BlockSpec((1, 1, D), wi),
            pl.BlockSpec((C, 1), lambda e: (jnp.minimum(e, E - 1), 0)),
        ],
        out_specs=pl.BlockSpec((C, D), lambda e: (e, 0)),
        out_shape=jax.ShapeDtypeStruct((EC + C, D), jnp.float32),
        compiler_params=pltpu.CompilerParams(
            dimension_semantics=("arbitrary",),
        ),
    )(xe, eW1, eb1.reshape(E, 1, FF), eW2, eb2.reshape(E, 1, D),
      disp_gate)


def kernel(x, rW1, rb1, rW2, rb2, eW1, eb1, eW2, eb2):
    i1, i2, g1, g2 = _router(x, rW1, rb1, rW2, rb2)
    s1, s2 = _positions(i1, i2)
    disp_tok, disp_gate = _disp_table(s1, s2, g1, g2)
    xe = _sc_gather(x, disp_tok.reshape(EC))
    ye = _ffn(xe, eW1, eb1, eW2, eb2, disp_gate)
    out = _sc_combine(ye, s1.reshape(T), s2.reshape(T))
    return out
